# baseline (speedup 1.0000x reference)
import jax
import jax.numpy as jnp
from jax import lax
from jax.experimental import pallas as pl
from jax.experimental.pallas import tpu as pltpu
from jax.experimental.pallas import tpu_sc as plsc

EMBED_DIM = 300
N_EDGES = 160000

QD = 4 * EMBED_DIM
NQ = N_EDGES // 4

NC = 2
NS = 16
NW = NC * NS
PER_W = NQ // NW
QCHUNK = 50
NCHUNK = PER_W // QCHUNK


def _sc_body(stable_hbm, sidx_hbm, out_hbm, sidx_v, r0, r1, g0, g1, s0, s1):
    wid = lax.axis_index("s") * NC + lax.axis_index("c")
    base = wid * PER_W
    pltpu.sync_copy(sidx_hbm.at[wid], sidx_v)

    rows = (r0, r1)
    gsem = (g0, g1)
    ssem = (s0, s1)
    stores = [None, None]

    pltpu.async_copy(stable_hbm.at[sidx_v.at[0]], r0, g0)
    for c in range(NCHUNK):
        b = c % 2
        pltpu.make_async_copy(stable_hbm.at[sidx_v.at[c]], rows[b], gsem[b]).wait()
        if c + 1 < NCHUNK:
            nb = (c + 1) % 2
            if stores[nb] is not None:
                stores[nb].wait()
            pltpu.async_copy(stable_hbm.at[sidx_v.at[c + 1]], rows[nb], gsem[nb])
        stores[b] = pltpu.async_copy(
            rows[b], out_hbm.at[pl.ds(base + c * QCHUNK, QCHUNK)], ssem[b]
        )
    stores[0].wait()
    stores[1].wait()


def kernel(tensor, table):
    table = table.astype(jnp.float32)
    idx = tensor.astype(jnp.int32)

    digits = jax.lax.broadcasted_iota(jnp.int32, (256, 4), 0)
    shifts = jnp.array([6, 4, 2, 0], jnp.int32)
    combo = (digits >> shifts[None, :]) & 3
    stable = jnp.take(table, combo.reshape(-1), axis=0).reshape(256, QD)

    q = idx.reshape(NQ, 4)
    sidx = ((q[:, 0] * 4 + q[:, 1]) * 4 + q[:, 2]) * 4 + q[:, 3]
    sidx = sidx.reshape(NW, NCHUNK, QCHUNK)

    mesh = plsc.VectorSubcoreMesh(
        core_axis_name="c", subcore_axis_name="s", num_cores=NC, num_subcores=NS
    )
    run = pl.kernel(
        _sc_body,
        out_type=jax.ShapeDtypeStruct((NQ, QD), jnp.float32),
        mesh=mesh,
        scratch_types=[
            pltpu.VMEM((NCHUNK, QCHUNK), jnp.int32),
            pltpu.VMEM((QCHUNK, QD), jnp.float32),
            pltpu.VMEM((QCHUNK, QD), jnp.float32),
            pltpu.SemaphoreType.DMA,
            pltpu.SemaphoreType.DMA,
            pltpu.SemaphoreType.DMA,
            pltpu.SemaphoreType.DMA,
        ],
        compiler_params=pltpu.CompilerParams(use_tc_tiling_on_sc=False),
    )
    out = run(stable, sidx)
    return out.reshape(N_EDGES, EMBED_DIM)


# speedup vs baseline: 1.1472x; 1.1472x over previous
import jax
import jax.numpy as jnp
from jax import lax
from jax.experimental import pallas as pl
from jax.experimental.pallas import tpu as pltpu
from jax.experimental.pallas import tpu_sc as plsc

EMBED_DIM = 300
N_EDGES = 160000

QD = 4 * EMBED_DIM
NQ = N_EDGES // 4

NC = 2
NS = 16
NW = NC * NS
PER_W = NQ // NW
QCHUNK = 25
NCHUNK = PER_W // QCHUNK


def _sc_body(stable_hbm, sidx_hbm, out_hbm, t_sh, sidx_v, r0, r1, g0, g1, s0, s1):
    wid = lax.axis_index("s") * NC + lax.axis_index("c")
    base = wid * PER_W
    @pl.when(lax.axis_index("s") == 0)
    def _():
        pltpu.sync_copy(stable_hbm, t_sh)
    pltpu.sync_copy(sidx_hbm.at[wid], sidx_v)
    plsc.subcore_barrier()

    rows = (r0, r1)
    gsem = (g0, g1)
    ssem = (s0, s1)
    stores = [None, None]

    pltpu.async_copy(t_sh.at[sidx_v.at[0]], r0, g0)
    for c in range(NCHUNK):
        b = c % 2
        pltpu.make_async_copy(t_sh.at[sidx_v.at[c]], rows[b], gsem[b]).wait()
        if c + 1 < NCHUNK:
            nb = (c + 1) % 2
            if stores[nb] is not None:
                stores[nb].wait()
            pltpu.async_copy(t_sh.at[sidx_v.at[c + 1]], rows[nb], gsem[nb])
        stores[b] = pltpu.async_copy(
            rows[b], out_hbm.at[pl.ds(base + c * QCHUNK, QCHUNK)], ssem[b]
        )
    stores[0].wait()
    stores[1].wait()


def kernel(tensor, table):
    table = table.astype(jnp.float32)
    idx = tensor.astype(jnp.int32)

    digits = jax.lax.broadcasted_iota(jnp.int32, (256, 4), 0)
    shifts = jnp.array([6, 4, 2, 0], jnp.int32)
    combo = (digits >> shifts[None, :]) & 3
    stable = jnp.take(table, combo.reshape(-1), axis=0).reshape(256, QD)

    q = idx.reshape(NQ, 4)
    sidx = ((q[:, 0] * 4 + q[:, 1]) * 4 + q[:, 2]) * 4 + q[:, 3]
    sidx = sidx.reshape(NW, NCHUNK, QCHUNK)

    mesh = plsc.VectorSubcoreMesh(
        core_axis_name="c", subcore_axis_name="s", num_cores=NC, num_subcores=NS
    )
    run = pl.kernel(
        _sc_body,
        out_type=jax.ShapeDtypeStruct((NQ, QD), jnp.float32),
        mesh=mesh,
        scratch_types=[
            pltpu.VMEM_SHARED((256, QD), jnp.float32),
            pltpu.VMEM((NCHUNK, QCHUNK), jnp.int32),
            pltpu.VMEM((QCHUNK, QD), jnp.float32),
            pltpu.VMEM((QCHUNK, QD), jnp.float32),
            pltpu.SemaphoreType.DMA,
            pltpu.SemaphoreType.DMA,
            pltpu.SemaphoreType.DMA,
            pltpu.SemaphoreType.DMA,
        ],
        compiler_params=pltpu.CompilerParams(use_tc_tiling_on_sc=False),
    )
    out = run(stable, sidx)
    return out.reshape(N_EDGES, EMBED_DIM)
